# baseline (speedup 1.0000x reference)
import functools

import jax
import jax.numpy as jnp
from jax import lax
from jax.experimental import pallas as pl
from jax.experimental.pallas import tpu as pltpu
from jax.experimental.pallas import tpu_sc as plsc

M = 8
KS = 256
DS = 16

NC = 2
NS = 16
NW = NC * NS


def _encode_body(vecs_ref, cwt_ref, codes_ref):
    b = vecs_ref.shape[0]
    cols = []
    for m in range(M):
        sub = vecs_ref[:, m * DS:(m + 1) * DS]
        cwt = cwt_ref[m * DS:(m + 1) * DS, :]
        xc = jnp.dot(sub, cwt, preferred_element_type=jnp.float32)
        c2h = 0.5 * jnp.sum(cwt * cwt, axis=0, keepdims=True)
        score = c2h - xc
        minval = jnp.min(score, axis=1, keepdims=True)
        lane = lax.broadcasted_iota(jnp.int32, (b, KS), 1).astype(jnp.float32)
        idx = jnp.min(jnp.where(score == minval, lane, float(KS)),
                      axis=1, keepdims=True)
        cols.append(idx + m * KS)
    codes_ref[:, :] = jnp.concatenate(cols, axis=1).astype(jnp.int32)


def _encode(vecs, cwt, block_b, row_off, rows):
    grid = (rows // block_b,)
    off_b = row_off // block_b
    return pl.pallas_call(
        _encode_body,
        grid=grid,
        in_specs=[
            pl.BlockSpec((block_b, M * DS), lambda i: (i + off_b, 0)),
            pl.BlockSpec((M * DS, KS), lambda i: (0, 0)),
        ],
        out_specs=pl.BlockSpec((block_b, M), lambda i: (i, 0)),
        out_shape=jax.ShapeDtypeStruct((rows, M), jnp.int32),
    )(vecs, cwt)


def _make_decode(total, chunk):
    b_per_w = total // NW
    n_chunks = b_per_w // chunk
    mesh = plsc.VectorSubcoreMesh(
        core_axis_name="c", subcore_axis_name="s",
        num_cores=NC, num_subcores=NS)

    @functools.partial(
        pl.kernel,
        out_type=jax.ShapeDtypeStruct((total, DS), jnp.float32),
        mesh=mesh,
        scratch_types=[
            pltpu.VMEM((chunk,), jnp.int32),
            pltpu.VMEM((chunk, DS), jnp.float32),
            pltpu.SemaphoreType.DMA,
        ],
        compiler_params=pltpu.CompilerParams(use_tc_tiling_on_sc=False),
    )
    def decode(table_hbm, idx_hbm, out_hbm, idx_v, rows_v, sem):
        wid = lax.axis_index("s") * NC + lax.axis_index("c")
        base = wid * b_per_w
        for c in range(n_chunks):
            off = base + c * chunk
            pltpu.sync_copy(idx_hbm.at[pl.ds(off, chunk)], idx_v)
            pltpu.async_copy(table_hbm.at[idx_v], rows_v, sem).wait()
            pltpu.sync_copy(rows_v, out_hbm.at[pl.ds(off, chunk)])

    return decode


def kernel(vecs, codewords):
    n, d = vecs.shape
    m_, ks_, ds_ = codewords.shape
    cwt = codewords.transpose(0, 2, 1).reshape(m_ * ds_, ks_)
    table = codewords.reshape(m_ * ks_, ds_)
    n_chunks = 4
    nc = n // n_chunks
    decode = _make_decode(nc * m_, nc * m_ // NW)
    parts = []
    for i in range(n_chunks):
        codes = _encode(vecs, cwt, 1024, i * nc, nc)
        parts.append(decode(table, codes.reshape(nc * m_)))
    return jnp.concatenate(parts, axis=0).reshape(n, d)


# speedup vs baseline: 2.3957x; 2.3957x over previous
import functools

import jax
import jax.numpy as jnp
from jax import lax
from jax.experimental import pallas as pl
from jax.experimental.pallas import tpu as pltpu
from jax.experimental.pallas import tpu_sc as plsc

M = 8
KS = 256
DS = 16

NC = 2
NS = 16
NW = NC * NS


def _encode_body(vecs_ref, cwt_ref, codes_ref):
    b = vecs_ref.shape[0]

    def _score(m):
        sub = vecs_ref[:, m * DS:(m + 1) * DS]
        cwt = cwt_ref[m * DS:(m + 1) * DS, :]
        xc = jnp.dot(sub, cwt, preferred_element_type=jnp.float32)
        c2h = 0.5 * jnp.sum(cwt * cwt, axis=0, keepdims=True)
        return c2h - xc

    lane = lax.broadcasted_iota(jnp.int32, (b, 128), 1).astype(jnp.float32)

    def _argmin(score, m):
        s0 = score[:, :128]
        s1 = score[:, 128:]
        t = jnp.minimum(s0, s1)
        minval = jnp.min(t, axis=1, keepdims=True)
        lane2 = jnp.where(s1 < s0, lane + 128.0, lane)
        idx = jnp.min(jnp.where(t == minval, lane2, float(2 * KS)),
                      axis=1, keepdims=True)
        return idx + m * KS

    cols = []
    score_prev = _score(0)
    for m in range(1, M):
        score_next = _score(m)
        cols.append(_argmin(score_prev, m - 1))
        score_prev = score_next
    cols.append(_argmin(score_prev, M - 1))
    codes_ref[:, :] = jnp.concatenate(cols, axis=1).astype(jnp.int32)


def _encode(vecs, cwt, block_b):
    n = vecs.shape[0]
    grid = (n // block_b,)
    return pl.pallas_call(
        _encode_body,
        grid=grid,
        in_specs=[
            pl.BlockSpec((block_b, M * DS), lambda i: (i, 0)),
            pl.BlockSpec((M * DS, KS), lambda i: (0, 0)),
        ],
        out_specs=pl.BlockSpec((block_b, M), lambda i: (i, 0)),
        out_shape=jax.ShapeDtypeStruct((n, M), jnp.int32),
    )(vecs, cwt)


def _make_decode(total, chunk):
    b_per_w = total // NW
    n_chunks = b_per_w // chunk
    mesh = plsc.VectorSubcoreMesh(
        core_axis_name="c", subcore_axis_name="s",
        num_cores=NC, num_subcores=NS)

    @functools.partial(
        pl.kernel,
        out_type=jax.ShapeDtypeStruct((total, DS), jnp.float32),
        mesh=mesh,
        scratch_types=[
            pltpu.VMEM((chunk,), jnp.int32),
            pltpu.VMEM((chunk, DS), jnp.float32),
            pltpu.SemaphoreType.DMA,
        ],
        compiler_params=pltpu.CompilerParams(use_tc_tiling_on_sc=False),
    )
    def decode(table_hbm, idx_hbm, out_hbm, idx_v, rows_v, sem):
        wid = lax.axis_index("s") * NC + lax.axis_index("c")
        base = wid * b_per_w
        for c in range(n_chunks):
            off = base + c * chunk
            pltpu.sync_copy(idx_hbm.at[pl.ds(off, chunk)], idx_v)
            pltpu.async_copy(table_hbm.at[idx_v], rows_v, sem).wait()
            pltpu.sync_copy(rows_v, out_hbm.at[pl.ds(off, chunk)])

    return decode


def kernel(vecs, codewords):
    n, d = vecs.shape
    m_, ks_, ds_ = codewords.shape
    cwt = codewords.transpose(0, 2, 1).reshape(m_ * ds_, ks_)
    table = codewords.reshape(m_ * ks_, ds_)
    codes = _encode(vecs, cwt, block_b=1024)
    flat_codes = codes.reshape(n * m_)
    rows = _make_decode(n * m_, 2048)(table, flat_codes)
    return rows.reshape(n, d)


# speedup vs baseline: 2.7071x; 1.1300x over previous
import functools

import jax
import jax.numpy as jnp
from jax import lax
from jax.experimental import pallas as pl
from jax.experimental.pallas import tpu as pltpu
from jax.experimental.pallas import tpu_sc as plsc

M = 8
KS = 256
DS = 16

NC = 2
NS = 16
NW = NC * NS


def _encode_body(vecs_ref, cwt_ref, codes_ref):
    b = vecs_ref.shape[0]

    def _score(m):
        sub = vecs_ref[:, m * DS:(m + 1) * DS]
        cwt = cwt_ref[m * DS:(m + 1) * DS, :]
        xc = jnp.dot(sub, cwt, preferred_element_type=jnp.float32)
        c2h = 0.5 * jnp.sum(cwt * cwt, axis=0, keepdims=True)
        return c2h - xc

    lane = lax.broadcasted_iota(jnp.int32, (b, 128), 1).astype(jnp.float32)

    def _argmin(score, m):
        s0 = score[:, :128]
        s1 = score[:, 128:]
        t = jnp.minimum(s0, s1)
        minval = jnp.min(t, axis=1, keepdims=True)
        lane2 = jnp.where(s1 < s0, lane + 128.0, lane)
        idx = jnp.min(jnp.where(t == minval, lane2, float(2 * KS)),
                      axis=1, keepdims=True)
        return idx + m * KS

    cols = []
    score_prev = _score(0)
    for m in range(1, M):
        score_next = _score(m)
        cols.append(_argmin(score_prev, m - 1))
        score_prev = score_next
    cols.append(_argmin(score_prev, M - 1))
    codes_ref[:, :] = jnp.concatenate(cols, axis=1).astype(jnp.int32)


def _encode(vecs, cwt, block_b):
    n = vecs.shape[0]
    grid = (n // block_b,)
    return pl.pallas_call(
        _encode_body,
        grid=grid,
        in_specs=[
            pl.BlockSpec((block_b, M * DS), lambda i: (i, 0)),
            pl.BlockSpec((M * DS, KS), lambda i: (0, 0)),
        ],
        out_specs=pl.BlockSpec((block_b, M), lambda i: (i, 0)),
        out_shape=jax.ShapeDtypeStruct((n, M), jnp.int32),
    )(vecs, cwt)


def _make_decode(total, chunk):
    b_per_w = total // NW
    n_chunks = b_per_w // chunk
    mesh = plsc.VectorSubcoreMesh(
        core_axis_name="c", subcore_axis_name="s",
        num_cores=NC, num_subcores=NS)

    @functools.partial(
        pl.kernel,
        out_type=jax.ShapeDtypeStruct((total, DS), jnp.float32),
        mesh=mesh,
        scratch_types=[
            pltpu.VMEM((chunk,), jnp.int32),
            pltpu.VMEM((chunk, DS), jnp.float32),
            pltpu.SemaphoreType.DMA,
        ],
        compiler_params=pltpu.CompilerParams(use_tc_tiling_on_sc=False),
    )
    def decode(table_hbm, idx_hbm, out_hbm, idx_v, rows_v, sem):
        wid = lax.axis_index("s") * NC + lax.axis_index("c")
        base = wid * b_per_w
        for c in range(n_chunks):
            off = base + c * chunk
            pltpu.sync_copy(idx_hbm.at[pl.ds(off, chunk)], idx_v)
            pltpu.async_copy(table_hbm.at[idx_v], rows_v, sem).wait()
            pltpu.sync_copy(rows_v, out_hbm.at[pl.ds(off, chunk)])

    return decode


def kernel(vecs, codewords):
    n, d = vecs.shape
    m_, ks_, ds_ = codewords.shape
    cwt = codewords.transpose(0, 2, 1).reshape(m_ * ds_, ks_)
    table = codewords.reshape(m_ * ks_, ds_)
    codes = _encode(vecs, cwt, block_b=2048)
    flat_codes = codes.reshape(n * m_)
    rows = _make_decode(n * m_, 2048)(table, flat_codes)
    return rows.reshape(n, d)


# speedup vs baseline: 2.7528x; 1.0169x over previous
import functools

import jax
import jax.numpy as jnp
from jax import lax
from jax.experimental import pallas as pl
from jax.experimental.pallas import tpu as pltpu
from jax.experimental.pallas import tpu_sc as plsc

M = 8
KS = 256
DS = 16

NC = 2
NS = 16
NW = NC * NS


def _encode_body(vecs_ref, cwt_ref, codes_ref):
    b = vecs_ref.shape[0]

    def _score(m):
        sub = vecs_ref[:, m * DS:(m + 1) * DS]
        cwt = cwt_ref[m * DS:(m + 1) * DS, :]
        xc = jnp.dot(sub, cwt, preferred_element_type=jnp.float32)
        c2h = 0.5 * jnp.sum(cwt * cwt, axis=0, keepdims=True)
        return c2h - xc

    lane = lax.broadcasted_iota(jnp.int32, (b, 128), 1).astype(jnp.float32)

    def _argmin(score, m):
        s0 = score[:, :128]
        s1 = score[:, 128:]
        t = jnp.minimum(s0, s1)
        minval = jnp.min(t, axis=1, keepdims=True)
        lane2 = jnp.where(s1 < s0, lane + 128.0, lane)
        idx = jnp.min(jnp.where(t == minval, lane2, float(2 * KS)),
                      axis=1, keepdims=True)
        return idx + m * KS

    cols = []
    score_prev = _score(0)
    for m in range(1, M):
        score_next = _score(m)
        cols.append(_argmin(score_prev, m - 1))
        score_prev = score_next
    cols.append(_argmin(score_prev, M - 1))
    codes_ref[:, :] = jnp.concatenate(cols, axis=1).astype(jnp.int32)


def _encode(vecs, cwt, block_b):
    n = vecs.shape[0]
    grid = (n // block_b,)
    return pl.pallas_call(
        _encode_body,
        grid=grid,
        in_specs=[
            pl.BlockSpec((block_b, M * DS), lambda i: (i, 0)),
            pl.BlockSpec((M * DS, KS), lambda i: (0, 0)),
        ],
        out_specs=pl.BlockSpec((block_b, M), lambda i: (i, 0)),
        out_shape=jax.ShapeDtypeStruct((n, M), jnp.int32),
    )(vecs, cwt)


def _make_decode(total, chunk):
    b_per_w = total // NW
    n_chunks = b_per_w // chunk
    mesh = plsc.VectorSubcoreMesh(
        core_axis_name="c", subcore_axis_name="s",
        num_cores=NC, num_subcores=NS)

    @functools.partial(
        pl.kernel,
        out_type=jax.ShapeDtypeStruct((total, DS), jnp.float32),
        mesh=mesh,
        scratch_types=[
            pltpu.VMEM((chunk,), jnp.int32),
            pltpu.VMEM((chunk, DS), jnp.float32),
            pltpu.SemaphoreType.DMA,
        ],
        compiler_params=pltpu.CompilerParams(use_tc_tiling_on_sc=False),
    )
    def decode(table_hbm, idx_hbm, out_hbm, idx_v, rows_v, sem):
        wid = lax.axis_index("s") * NC + lax.axis_index("c")
        base = wid * b_per_w
        for c in range(n_chunks):
            off = base + c * chunk
            pltpu.sync_copy(idx_hbm.at[pl.ds(off, chunk)], idx_v)
            pltpu.async_copy(table_hbm.at[idx_v], rows_v, sem).wait()
            pltpu.sync_copy(rows_v, out_hbm.at[pl.ds(off, chunk)])

    return decode


def kernel(vecs, codewords):
    n, d = vecs.shape
    m_, ks_, ds_ = codewords.shape
    cwt = codewords.transpose(0, 2, 1).reshape(m_ * ds_, ks_)
    table = codewords.reshape(m_ * ks_, ds_)
    codes = _encode(vecs, cwt, block_b=4096)
    flat_codes = codes.reshape(n * m_)
    rows = _make_decode(n * m_, 2048)(table, flat_codes)
    return rows.reshape(n, d)
